# Initial kernel scaffold; baseline (speedup 1.0000x reference)
import jax
import jax.numpy as jnp
from jax.experimental import pallas as pl


def kernel(x, W1, gamma, beta, W2, b2):
    raise NotImplementedError("write your pallas kernel here")


# speedup vs baseline: 1.0031x; 1.0031x over previous
import functools

import jax
import jax.numpy as jnp
from jax.experimental import pallas as pl
from jax.experimental.pallas import tpu as pltpu

_TEMP = 0.1
_EPS = 1e-5
_TOPK = 8


def _router_kernel(x_ref, w1_ref, gamma_ref, beta_ref, w2_ref, b2_ref,
                   rw_ref, idx_ref, logits_ref, *, n_experts):
    h = jax.lax.dot_general(
        x_ref[...].astype(jnp.bfloat16), w1_ref[...],
        (((1,), (1,)), ((), ())),
        preferred_element_type=jnp.float32,
    )
    mu = jnp.mean(h, axis=1, keepdims=True)
    var = jnp.mean((h - mu) ** 2, axis=1, keepdims=True)
    ln = (h - mu) * jax.lax.rsqrt(var + _EPS) * gamma_ref[...] + beta_ref[...]
    s = ln * jax.nn.sigmoid(ln)
    logits = jax.lax.dot_general(
        s.astype(jnp.bfloat16), w2_ref[...],
        (((1,), (1,)), ((), ())),
        preferred_element_type=jnp.float32,
    ) + b2_ref[...]
    logits_ref[...] = logits

    z = logits / _TEMP
    z = z - jnp.max(z, axis=1, keepdims=True)
    ez = jnp.exp(z)
    w = ez / jnp.sum(ez, axis=1, keepdims=True)

    tm = w.shape[0]
    ii = jax.lax.broadcasted_iota(jnp.int32, (tm, n_experts), 1)
    cur = w
    vals, idxs = [], []
    for _ in range(_TOPK):
        m = jnp.max(cur, axis=1, keepdims=True)
        j = jnp.min(jnp.where(cur == m, ii, n_experts), axis=1, keepdims=True)
        vals.append(m)
        idxs.append(j)
        cur = jnp.where(ii == j, -1.0, cur)
    rw_ref[...] = jnp.concatenate(vals, axis=1)
    idx_ref[...] = jnp.concatenate(idxs, axis=1)


def kernel(x, W1, gamma, beta, W2, b2):
    n_tok, h_dim = x.shape
    n_experts = W2.shape[0]
    tm = min(256, n_tok)
    n_i = n_tok // tm

    w1_bf = W1.astype(jnp.bfloat16)
    w2_bf = W2.astype(jnp.bfloat16)
    gamma2 = gamma.reshape(1, h_dim)
    beta2 = beta.reshape(1, h_dim)
    b22 = b2.reshape(1, n_experts)

    body = functools.partial(_router_kernel, n_experts=n_experts)
    rw, idx, logits = pl.pallas_call(
        body,
        grid=(n_i,),
        in_specs=[
            pl.BlockSpec((tm, h_dim), lambda i: (i, 0)),
            pl.BlockSpec((h_dim, h_dim), lambda i: (0, 0)),
            pl.BlockSpec((1, h_dim), lambda i: (0, 0)),
            pl.BlockSpec((1, h_dim), lambda i: (0, 0)),
            pl.BlockSpec((n_experts, h_dim), lambda i: (0, 0)),
            pl.BlockSpec((1, n_experts), lambda i: (0, 0)),
        ],
        out_specs=[
            pl.BlockSpec((tm, _TOPK), lambda i: (i, 0)),
            pl.BlockSpec((tm, _TOPK), lambda i: (i, 0)),
            pl.BlockSpec((tm, n_experts), lambda i: (i, 0)),
        ],
        out_shape=[
            jax.ShapeDtypeStruct((n_tok, _TOPK), jnp.float32),
            jax.ShapeDtypeStruct((n_tok, _TOPK), jnp.int32),
            jax.ShapeDtypeStruct((n_tok, n_experts), jnp.float32),
        ],
        compiler_params=pltpu.CompilerParams(
            dimension_semantics=("arbitrary",),
        ),
    )(x, w1_bf, gamma2, beta2, w2_bf, b22)
    return (rw, idx, logits)
